# Initial kernel scaffold; baseline (speedup 1.0000x reference)
import jax
import jax.numpy as jnp
from jax.experimental import pallas as pl


def kernel(x, nbr_fea, nbr_fea_idx, ln_scale, ln_bias, W1, b1, W2, b2):
    raise NotImplementedError("write your pallas kernel here")


# speedup vs baseline: 2.1015x; 2.1015x over previous
import functools

import jax
import jax.numpy as jnp
from jax import lax
from jax.experimental import pallas as pl
from jax.experimental.pallas import tpu as pltpu
from jax.experimental.pallas import tpu_sc as plsc


def _prep_body(x_ref, scale_ref, bias_ref, w1a_ref, w1b_ref, b1_ref,
               a_ref, g_ref):
    xb = x_ref[...]
    mean = jnp.mean(xb, axis=1, keepdims=True)
    var = jnp.mean(jnp.square(xb - mean), axis=1, keepdims=True)
    xn = (xb - mean) * lax.rsqrt(var + 1e-6) * scale_ref[...] + bias_ref[...]
    a_ref[...] = (jnp.dot(xn, w1a_ref[...], preferred_element_type=jnp.float32)
                  + b1_ref[...])
    g_ref[...] = jnp.dot(xn, w1b_ref[...], preferred_element_type=jnp.float32)


def _prep(x, ln_scale, ln_bias, w1a, w1b, b1, block_n):
    n, d = x.shape
    grid = n // block_n
    return pl.pallas_call(
        _prep_body,
        grid=(grid,),
        in_specs=[
            pl.BlockSpec((block_n, d), lambda i: (i, 0)),
            pl.BlockSpec((1, d), lambda i: (0, 0)),
            pl.BlockSpec((1, d), lambda i: (0, 0)),
            pl.BlockSpec((d, d), lambda i: (0, 0)),
            pl.BlockSpec((d, d), lambda i: (0, 0)),
            pl.BlockSpec((1, d), lambda i: (0, 0)),
        ],
        out_specs=[
            pl.BlockSpec((block_n, d), lambda i: (i, 0)),
            pl.BlockSpec((block_n, d), lambda i: (i, 0)),
        ],
        out_shape=[
            jax.ShapeDtypeStruct((n, d), jnp.float32),
            jax.ShapeDtypeStruct((n, d), jnp.float32),
        ],
    )(x, ln_scale, ln_bias, w1a, w1b, b1)


def _make_gather(n_edges, d, chunk):
    info = plsc.get_sparse_core_info()
    nw = info.num_cores * info.num_subcores
    per_w = n_edges // nw
    n_ch = per_w // chunk
    mesh = plsc.VectorSubcoreMesh(core_axis_name="c", subcore_axis_name="s")

    @functools.partial(
        pl.kernel,
        mesh=mesh,
        out_type=jax.ShapeDtypeStruct((n_edges, d), jnp.float32),
        scratch_types=[
            pltpu.VMEM((chunk,), jnp.int32),
            pltpu.VMEM((chunk, d), jnp.float32),
            pltpu.SemaphoreType.DMA,
        ],
    )
    def gather_k(table_hbm, idx_hbm, out_hbm, idx_v, rows_v, sem):
        c = lax.axis_index("c")
        s = lax.axis_index("s")
        wid = s * info.num_cores + c
        base0 = wid * per_w

        def body(j, carry):
            base = pl.multiple_of(base0 + j * chunk, 8)
            pltpu.sync_copy(idx_hbm.at[pl.ds(base, chunk)], idx_v)
            pltpu.async_copy(table_hbm.at[idx_v], rows_v, sem).wait()
            pltpu.sync_copy(rows_v, out_hbm.at[pl.ds(base, chunk)])
            return carry

        lax.fori_loop(0, n_ch, body, 0)

    return gather_k


def _main_body(m, gg_ref, nbr_ref, a_ref, x_ref, w1c_ref, w2_ref, b2_ref,
               out_ref):
    bn, d = a_ref.shape
    r = bn * m
    f = jnp.dot(nbr_ref[...], w1c_ref[...], preferred_element_type=jnp.float32)
    a_rep = jnp.broadcast_to(a_ref[...][:, None, :], (bn, m, d)).reshape(r, d)
    pre = gg_ref[...] + f + a_rep
    s = pre * (1.0 / (1.0 + jnp.exp(-pre)))
    h = jnp.sum(s.reshape(bn, m, d), axis=1)
    out_ref[...] = (x_ref[...]
                    + jnp.dot(h, w2_ref[...], preferred_element_type=jnp.float32)
                    + b2_ref[...])


def _main(gg, nbr_flat, a, x, w1c, w2, b2m, m, block_n):
    n, d = x.shape
    e = nbr_flat.shape[1]
    grid = n // block_n
    br = block_n * m
    return pl.pallas_call(
        functools.partial(_main_body, m),
        grid=(grid,),
        in_specs=[
            pl.BlockSpec((br, d), lambda i: (i, 0)),
            pl.BlockSpec((br, e), lambda i: (i, 0)),
            pl.BlockSpec((block_n, d), lambda i: (i, 0)),
            pl.BlockSpec((block_n, d), lambda i: (i, 0)),
            pl.BlockSpec((e, d), lambda i: (0, 0)),
            pl.BlockSpec((d, d), lambda i: (0, 0)),
            pl.BlockSpec((1, d), lambda i: (0, 0)),
        ],
        out_specs=pl.BlockSpec((block_n, d), lambda i: (i, 0)),
        out_shape=jax.ShapeDtypeStruct((n, d), jnp.float32),
    )(gg, nbr_flat, a, x, w1c, w2, b2m)


def kernel(x, nbr_fea, nbr_fea_idx, ln_scale, ln_bias, W1, b1, W2, b2):
    n, d = x.shape
    m = nbr_fea_idx.shape[1]
    e = nbr_fea.shape[2]

    w1a = W1[:d]
    w1b = W1[d:2 * d]
    w1c = W1[2 * d:]

    a, g = _prep(x, ln_scale.reshape(1, d), ln_bias.reshape(1, d),
                 w1a, w1b, b1.reshape(1, d), block_n=2000)

    idx_flat = nbr_fea_idx.reshape(n * m).astype(jnp.int32)
    gg = _make_gather(n * m, d, chunk=80)(g, idx_flat)

    nbr_flat = nbr_fea.reshape(n * m, e)
    b2m = (b2 * float(m)).reshape(1, d)
    return _main(gg, nbr_flat, a, x, w1c, W2, b2m, m, block_n=200)


# speedup vs baseline: 3.0598x; 1.4560x over previous
import functools

import jax
import jax.numpy as jnp
from jax import lax
from jax.experimental import pallas as pl
from jax.experimental.pallas import tpu as pltpu
from jax.experimental.pallas import tpu_sc as plsc


def _prep_body(x_ref, scale_ref, bias_ref, w1a_ref, w1b_ref, b1_ref,
               a_ref, g_ref):
    xb = x_ref[...]
    mean = jnp.mean(xb, axis=1, keepdims=True)
    var = jnp.mean(jnp.square(xb - mean), axis=1, keepdims=True)
    xn = (xb - mean) * lax.rsqrt(var + 1e-6) * scale_ref[...] + bias_ref[...]
    a_ref[...] = (jnp.dot(xn, w1a_ref[...], preferred_element_type=jnp.float32)
                  + b1_ref[...])
    g_ref[...] = jnp.dot(xn, w1b_ref[...], preferred_element_type=jnp.float32)


def _prep(x, ln_scale, ln_bias, w1a, w1b, b1, block_n):
    n, d = x.shape
    grid = n // block_n
    return pl.pallas_call(
        _prep_body,
        grid=(grid,),
        in_specs=[
            pl.BlockSpec((block_n, d), lambda i: (i, 0)),
            pl.BlockSpec((1, d), lambda i: (0, 0)),
            pl.BlockSpec((1, d), lambda i: (0, 0)),
            pl.BlockSpec((d, d), lambda i: (0, 0)),
            pl.BlockSpec((d, d), lambda i: (0, 0)),
            pl.BlockSpec((1, d), lambda i: (0, 0)),
        ],
        out_specs=[
            pl.BlockSpec((block_n, d), lambda i: (i, 0)),
            pl.BlockSpec((block_n, d), lambda i: (i, 0)),
        ],
        out_shape=[
            jax.ShapeDtypeStruct((n, d), jnp.float32),
            jax.ShapeDtypeStruct((n, d), jnp.float32),
        ],
    )(x, ln_scale, ln_bias, w1a, w1b, b1)


def _make_gather(n_edges, d, chunk):
    info = plsc.get_sparse_core_info()
    nw = info.num_cores * info.num_subcores
    per_w = n_edges // nw
    n_ch = per_w // chunk
    mesh = plsc.VectorSubcoreMesh(core_axis_name="c", subcore_axis_name="s")

    @functools.partial(
        pl.kernel,
        mesh=mesh,
        out_type=jax.ShapeDtypeStruct((n_edges, d), jnp.float32),
        scratch_types=[
            pltpu.VMEM((chunk,), jnp.int32),
            pltpu.VMEM((chunk,), jnp.int32),
            pltpu.VMEM((chunk, d), jnp.float32),
            pltpu.VMEM((chunk, d), jnp.float32),
            pltpu.SemaphoreType.DMA,
            pltpu.SemaphoreType.DMA,
            pltpu.SemaphoreType.DMA,
            pltpu.SemaphoreType.DMA,
            pltpu.SemaphoreType.DMA,
            pltpu.SemaphoreType.DMA,
        ],
    )
    def gather_k(table_hbm, idx_hbm, out_hbm, idx0, idx1, rows0, rows1,
                 si0, si1, sg0, sg1, ss0, ss1):
        c = lax.axis_index("c")
        s = lax.axis_index("s")
        wid = s * info.num_cores + c
        base0 = wid * per_w
        idx_v = (idx0, idx1)
        rows_v = (rows0, rows1)
        sem_i = (si0, si1)
        sem_g = (sg0, sg1)
        sem_s = (ss0, ss1)

        def _idx_start(j, b):
            base = pl.multiple_of(base0 + j * chunk, 8)
            pltpu.async_copy(idx_hbm.at[pl.ds(base, chunk)], idx_v[b],
                             sem_i[b])

        _idx_start(0, 0)
        _idx_start(1, 1)

        def body(t, carry):
            for b in (0, 1):
                j = 2 * t + b

                @pl.when(t > 0)
                def _():
                    pltpu.make_async_copy(
                        rows_v[b], out_hbm.at[pl.ds(0, chunk)],
                        sem_s[b]).wait()

                pltpu.make_async_copy(
                    idx_hbm.at[pl.ds(0, chunk)], idx_v[b], sem_i[b]).wait()
                pltpu.async_copy(table_hbm.at[idx_v[b]], rows_v[b],
                                 sem_g[b]).wait()

                @pl.when(j + 2 < n_ch)
                def _():
                    _idx_start(j + 2, b)

                base = pl.multiple_of(base0 + j * chunk, 8)
                pltpu.async_copy(rows_v[b], out_hbm.at[pl.ds(base, chunk)],
                                 sem_s[b])
            return carry

        lax.fori_loop(0, n_ch // 2, body, 0)

        for b in (0, 1):
            pltpu.make_async_copy(
                rows_v[b], out_hbm.at[pl.ds(0, chunk)], sem_s[b]).wait()

    return gather_k


def _main_body(m, gg_ref, nbr_ref, a_ref, x_ref, w1c_ref, w2_ref, b2_ref,
               out_ref):
    bn, d = a_ref.shape
    r = bn * m
    f = jnp.dot(nbr_ref[...], w1c_ref[...], preferred_element_type=jnp.float32)
    a_rep = jnp.broadcast_to(a_ref[...][:, None, :], (bn, m, d)).reshape(r, d)
    pre = gg_ref[...] + f + a_rep
    s = pre * (1.0 / (1.0 + jnp.exp(-pre)))
    h = jnp.sum(s.reshape(bn, m, d), axis=1)
    out_ref[...] = (x_ref[...]
                    + jnp.dot(h, w2_ref[...], preferred_element_type=jnp.float32)
                    + b2_ref[...])


def _main(gg, nbr_flat, a, x, w1c, w2, b2m, m, block_n):
    n, d = x.shape
    e = nbr_flat.shape[1]
    grid = n // block_n
    br = block_n * m
    return pl.pallas_call(
        functools.partial(_main_body, m),
        grid=(grid,),
        in_specs=[
            pl.BlockSpec((br, d), lambda i: (i, 0)),
            pl.BlockSpec((br, e), lambda i: (i, 0)),
            pl.BlockSpec((block_n, d), lambda i: (i, 0)),
            pl.BlockSpec((block_n, d), lambda i: (i, 0)),
            pl.BlockSpec((e, d), lambda i: (0, 0)),
            pl.BlockSpec((d, d), lambda i: (0, 0)),
            pl.BlockSpec((1, d), lambda i: (0, 0)),
        ],
        out_specs=pl.BlockSpec((block_n, d), lambda i: (i, 0)),
        out_shape=jax.ShapeDtypeStruct((n, d), jnp.float32),
    )(gg, nbr_flat, a, x, w1c, w2, b2m)


def kernel(x, nbr_fea, nbr_fea_idx, ln_scale, ln_bias, W1, b1, W2, b2):
    n, d = x.shape
    m = nbr_fea_idx.shape[1]
    e = nbr_fea.shape[2]

    w1a = W1[:d]
    w1b = W1[d:2 * d]
    w1c = W1[2 * d:]

    a, g = _prep(x, ln_scale.reshape(1, d), ln_bias.reshape(1, d),
                 w1a, w1b, b1.reshape(1, d), block_n=2000)

    idx_flat = nbr_fea_idx.reshape(n * m).astype(jnp.int32)
    gg = _make_gather(n * m, d, chunk=200)(g, idx_flat)

    nbr_flat = nbr_fea.reshape(n * m, e)
    b2m = (b2 * float(m)).reshape(1, d)
    return _main(gg, nbr_flat, a, x, w1c, W2, b2m, m, block_n=200)
